# Initial kernel scaffold; baseline (speedup 1.0000x reference)
import jax
import jax.numpy as jnp
from jax.experimental import pallas as pl


def kernel(inputs, embedding):
    raise NotImplementedError("write your pallas kernel here")


# speedup vs baseline: 3.5832x; 3.5832x over previous
import jax
import jax.numpy as jnp
from jax.experimental import pallas as pl
from jax.experimental.pallas import tpu as pltpu

_B, _D, _N = 16, 256, 1024
_NE = 1024


def _vq_body(x_ref, e_ref, et_ref, q_ref, idx_ref, loss_ref):
    b = pl.program_id(0)
    x = x_ref[0]
    e = e_ref[...]
    et = et_ref[...]

    scores = jax.lax.dot_general(
        et, x, (((1,), (0,)), ((), ())),
        preferred_element_type=jnp.float32)
    e_sq = jnp.sum(et * et, axis=1, keepdims=True)
    x_sq = jnp.sum(x * x, axis=0, keepdims=True)
    dist = (x_sq + e_sq) - 2.0 * scores

    minval = jnp.min(dist, axis=0, keepdims=True)
    iota = jax.lax.broadcasted_iota(jnp.int32, (_NE, _N), 0)
    idx = jnp.min(jnp.where(dist == minval, iota, _NE), axis=0,
                  keepdims=True)
    idx_ref[0] = idx

    onehot = jnp.where(iota == idx, 1.0, 0.0)
    q = jax.lax.dot_general(
        e, onehot, (((1,), (0,)), ((), ())),
        preferred_element_type=jnp.float32)
    q_ref[0] = q

    diff = q - x
    part = jnp.sum(diff * diff)

    @pl.when(b == 0)
    def _():
        loss_ref[0, 0] = part

    @pl.when(b > 0)
    def _():
        loss_ref[0, 0] = loss_ref[0, 0] + part


def kernel(inputs, embedding):
    emb_t = embedding.T

    q, idx, loss_sum = pl.pallas_call(
        _vq_body,
        grid=(_B,),
        in_specs=[
            pl.BlockSpec((1, _D, _N), lambda b: (b, 0, 0)),
            pl.BlockSpec((_D, _NE), lambda b: (0, 0)),
            pl.BlockSpec((_NE, _D), lambda b: (0, 0)),
        ],
        out_specs=[
            pl.BlockSpec((1, _D, _N), lambda b: (b, 0, 0)),
            pl.BlockSpec((1, 1, _N), lambda b: (b, 0, 0)),
            pl.BlockSpec((1, 1), lambda b: (0, 0),
                         memory_space=pltpu.SMEM),
        ],
        out_shape=[
            jax.ShapeDtypeStruct((_B, _D, _N), jnp.float32),
            jax.ShapeDtypeStruct((_B, 1, _N), jnp.int32),
            jax.ShapeDtypeStruct((1, 1), jnp.float32),
        ],
    )(inputs, embedding, emb_t)

    loss = loss_sum[0, 0] / jnp.float32(_B * _D * _N)
    return (q, idx.reshape(_B, _N), loss)
